# baseline (speedup 1.0000x reference)
import functools

import jax
import jax.numpy as jnp
from jax import lax
from jax.experimental import pallas as pl
from jax.experimental.pallas import tpu as pltpu
from jax.experimental.pallas import tpu_sc as plsc

N = 10000
D = 128
H = 2
E = 160000
RN = 1000
R = 3
NLAYERS = 2

LANES = 16
NTILES = 32
ROWS = 10112
NPB = ROWS // NTILES
SCANC = 4000
NSCAN = E // SCANC
SGRP = SCANC // LANES
PROC = 80
SCAP = 112


def _hal_body(x_ref, w_ref, asrc_ref, adst_ref, h_ref, al_ref):
    x = x_ref[...]
    for h in range(H):
        hh = jnp.dot(x, w_ref[h], preferred_element_type=jnp.float32)
        h_ref[:, h * D:(h + 1) * D] = hh
        al_ref[:, h:h + 1] = lax.dot_general(
            hh, asrc_ref[h:h + 1, :], (((1,), (1,)), ((), ())),
            preferred_element_type=jnp.float32)
        al_ref[:, H + h:H + h + 1] = lax.dot_general(
            hh, adst_ref[h:h + 1, :], (((1,), (1,)), ((), ())),
            preferred_element_type=jnp.float32)


def _hidden_al(x, gw, gas, gad):
    bn = 1000
    return pl.pallas_call(
        _hal_body,
        grid=(N // bn,),
        in_specs=[pl.BlockSpec((bn, D), lambda i: (i, 0)),
                  pl.BlockSpec((H, D, D), lambda i: (0, 0, 0)),
                  pl.BlockSpec((H, D), lambda i: (0, 0)),
                  pl.BlockSpec((H, D), lambda i: (0, 0))],
        out_specs=[pl.BlockSpec((bn, H * D), lambda i: (i, 0)),
                   pl.BlockSpec((bn, 2 * H), lambda i: (i, 0))],
        out_shape=[jax.ShapeDtypeStruct((N, H * D), jnp.float32),
                   jax.ShapeDtypeStruct((N, 2 * H), jnp.float32)],
    )(x, gw, gas, gad)


def _sc_edge_agg(hcat, als0, ald0, als1, ald1, src, dst):
    mesh = plsc.VectorSubcoreMesh(core_axis_name="c", subcore_axis_name="s")

    @functools.partial(
        pl.kernel,
        mesh=mesh,
        out_type=(jax.ShapeDtypeStruct((H, ROWS, D), jnp.float32),
                  jax.ShapeDtypeStruct((ROWS, LANES), jnp.float32)),
        compiler_params=pltpu.CompilerParams(needs_layout_passes=False,
                                             use_tc_tiling_on_sc=False),
        scratch_types=[
            pltpu.VMEM((NPB, D), jnp.float32),
            pltpu.VMEM((NPB, D), jnp.float32),
            pltpu.VMEM((NPB, LANES), jnp.float32),
            pltpu.VMEM((SCANC,), jnp.int32),
            pltpu.VMEM((SCANC,), jnp.int32),
            pltpu.VMEM((SCAP,), jnp.int32),
            pltpu.VMEM((SCAP,), jnp.int32),
            pltpu.VMEM((PROC,), jnp.int32),
            pltpu.VMEM((PROC,), jnp.float32),
            pltpu.VMEM((PROC,), jnp.float32),
            pltpu.VMEM((PROC,), jnp.float32),
            pltpu.VMEM((PROC,), jnp.float32),
            pltpu.VMEM((PROC,), jnp.float32),
            pltpu.VMEM((PROC,), jnp.float32),
            pltpu.VMEM((PROC, H * D), jnp.float32),
            pltpu.SemaphoreType.DMA,
            pltpu.SemaphoreType.DMA,
            pltpu.SemaphoreType.DMA,
        ],
    )
    def k(h_hbm, als0_hbm, ald0_hbm, als1_hbm, ald1_hbm, src_hbm, dst_hbm,
          pay_hbm, wsum_hbm,
          acc0, acc1, accw, srcscan, dstscan, ssrc, sdst, dlb,
          w0b, w1b, av0, ad0, av1, ad1, rowsb, semr, sema, semi):
        cid = lax.axis_index("c")
        sid = lax.axis_index("s")
        wid = cid * 16 + sid
        lo = wid * NPB
        iota = lax.iota(jnp.int32, LANES)
        zf = jnp.zeros((LANES,), jnp.float32)
        e0 = (iota == 0).astype(jnp.float32)
        e1 = (iota == 1).astype(jnp.float32)

        def zacc(i, _):
            for dpart in range(D // LANES):
                acc0[i, pl.ds(dpart * LANES, LANES)] = zf
                acc1[i, pl.ds(dpart * LANES, LANES)] = zf
            accw[i, :] = zf
            return 0
        lax.fori_loop(0, NPB, zacc, 0)
        for gz in range(SCAP // LANES):
            ssrc[pl.ds(gz * LANES, LANES)] = jnp.zeros((LANES,), jnp.int32)
            sdst[pl.ds(gz * LANES, LANES)] = jnp.zeros((LANES,), jnp.int32)

        def process_batch(count):
            sref = ssrc.at[pl.ds(0, PROC)]
            dref = sdst.at[pl.ds(0, PROC)]
            gr = pltpu.async_copy(h_hbm.at[sref], rowsb, semr)
            g0 = pltpu.async_copy(als0_hbm.at[sref], av0, sema)
            g1 = pltpu.async_copy(ald0_hbm.at[dref], ad0, sema)
            g2 = pltpu.async_copy(als1_hbm.at[sref], av1, sema)
            g3 = pltpu.async_copy(ald1_hbm.at[dref], ad1, sema)
            g0.wait()
            g1.wait()
            g2.wait()
            g3.wait()
            for g in range(PROC // LANES):
                sl = pl.ds(g * LANES, LANES)
                valid = (g * LANES + iota) < count
                x0 = av0[sl] + ad0[sl]
                w0 = jnp.where(valid, jnp.exp(jnp.maximum(x0, 0.2 * x0)), 0.0)
                x1 = av1[sl] + ad1[sl]
                w1 = jnp.where(valid, jnp.exp(jnp.maximum(x1, 0.2 * x1)), 0.0)
                w0b[sl] = w0
                w1b[sl] = w1
                dl = sdst[sl] - lo
                dlb[sl] = jnp.minimum(jnp.maximum(dl, 0), NPB - 1)
            gr.wait()

            def accum(g, _):
                sl = pl.ds(g * LANES, LANES)
                dl16 = dlb[sl]
                w016 = w0b[sl]
                w116 = w1b[sl]
                for j in range(LANES):
                    i = g * LANES + j
                    dlj = dl16[j]
                    w0j = w016[j]
                    w1j = w116[j]
                    for dpart in range(D // LANES):
                        c = pl.ds(dpart * LANES, LANES)
                        r0 = rowsb[i, c]
                        acc0[dlj, c] = acc0[dlj, c] + w0j * r0
                        r1 = rowsb[i, pl.ds(D + dpart * LANES, LANES)]
                        acc1[dlj, c] = acc1[dlj, c] + w1j * r1
                    accw[dlj, :] = accw[dlj, :] + w0j * e0 + w1j * e1
                return 0
            lax.fori_loop(0, PROC // LANES, accum, 0)

        def scan_chunk(sc, staged):
            ls = pltpu.async_copy(src_hbm.at[pl.ds(sc * SCANC, SCANC)],
                                  srcscan, semi)
            ld = pltpu.async_copy(dst_hbm.at[pl.ds(sc * SCANC, SCANC)],
                                  dstscan, semi)
            ls.wait()
            ld.wait()

            def group(g, st):
                sl = pl.ds(g * LANES, LANES)
                didx = dstscan[sl]
                dloc = didx - lo
                m = (dloc >= 0) & (dloc < NPB)
                cnt = plsc.all_reduce_population_count(m)[0]
                sidx = srcscan[sl]
                plsc.store_compressed(ssrc.at[pl.ds(st, LANES)], sidx, mask=m)
                plsc.store_compressed(sdst.at[pl.ds(st, LANES)], didx, mask=m)
                st = st + cnt

                @pl.when(st >= PROC)
                def _():
                    process_batch(PROC)
                    ssrc[pl.ds(0, LANES)] = ssrc[pl.ds(PROC, LANES)]
                    sdst[pl.ds(0, LANES)] = sdst[pl.ds(PROC, LANES)]
                return jnp.where(st >= PROC, st - PROC, st)
            return lax.fori_loop(0, SGRP, group, staged)
        staged = lax.fori_loop(0, NSCAN, scan_chunk, 0)

        @pl.when(staged > 0)
        def _():
            process_batch(staged)

        pltpu.sync_copy(acc0, pay_hbm.at[0, pl.ds(lo, NPB)])
        pltpu.sync_copy(acc1, pay_hbm.at[1, pl.ds(lo, NPB)])
        pltpu.sync_copy(accw, wsum_hbm.at[pl.ds(lo, NPB)])

    return k(hcat, als0, ald0, als1, ald1, src, dst)


def _comb_body(p_ref, w_ref, o_ref):
    accm = None
    for h in range(H):
        num = p_ref[h]
        den = w_ref[:, h:h + 1]
        v = num / (den + 1e-16)
        e = jnp.where(v > 0, v, jnp.exp(jnp.minimum(v, 0.0)) - 1.0)
        accm = e if accm is None else accm + e
    m = accm * (1.0 / H)
    nrm = jnp.sqrt(jnp.sum(m * m, axis=1, keepdims=True))
    o_ref[...] = m / (nrm + 1e-12)


def _combine(pay, wsum):
    bn = 632
    return pl.pallas_call(
        _comb_body,
        grid=(ROWS // bn,),
        in_specs=[pl.BlockSpec((H, bn, D), lambda i: (0, i, 0)),
                  pl.BlockSpec((bn, LANES), lambda i: (i, 0))],
        out_specs=pl.BlockSpec((bn, D), lambda i: (i, 0)),
        out_shape=jax.ShapeDtypeStruct((ROWS, D), jnp.float32),
    )(pay, wsum)


_INV_SQRT_D = 0.08838834764831845


def _mha_body(x0_ref, x1_ref, x2_ref, wq_ref, wk_ref, wv_ref, o_ref):
    xs = [x0_ref[...], x1_ref[...], x2_ref[...]]
    for h in range(H):
        q = [jnp.dot(x, wq_ref[h], preferred_element_type=jnp.float32)
             for x in xs]
        kk = [jnp.dot(x, wk_ref[h], preferred_element_type=jnp.float32)
              for x in xs]
        vv = [jnp.dot(x, wv_ref[h], preferred_element_type=jnp.float32)
              for x in xs]
        osum = None
        for r in range(R):
            att = [jnp.sum(q[r] * kk[s], axis=1, keepdims=True) * _INV_SQRT_D
                   for s in range(R)]
            m = jnp.maximum(jnp.maximum(att[0], att[1]), att[2])
            ee = [jnp.exp(a - m) for a in att]
            den = ee[0] + ee[1] + ee[2]
            o_r = (ee[0] * vv[0] + ee[1] * vv[1] + ee[2] * vv[2]) / den
            osum = o_r if osum is None else osum + o_r
        o_ref[:, h * D:(h + 1) * D] = osum * (1.0 / R)


def _mha(x0, x1, x2, wq, wk, wv):
    bn = 1000
    return pl.pallas_call(
        _mha_body,
        grid=(N // bn,),
        in_specs=[pl.BlockSpec((bn, D), lambda i: (i, 0)),
                  pl.BlockSpec((bn, D), lambda i: (i, 0)),
                  pl.BlockSpec((bn, D), lambda i: (i, 0)),
                  pl.BlockSpec((H, D, D), lambda i: (0, 0, 0)),
                  pl.BlockSpec((H, D, D), lambda i: (0, 0, 0)),
                  pl.BlockSpec((H, D, D), lambda i: (0, 0, 0))],
        out_specs=pl.BlockSpec((bn, H * D), lambda i: (i, 0)),
        out_shape=jax.ShapeDtypeStruct((N, H * D), jnp.float32),
    )(x0, x1, x2, wq, wk, wv)


def _proj_body(adj_ref, emb_ref, fused_ref, w_ref, b_ref, o_ref):
    adj = adj_ref[...]
    rs = jnp.sum(adj, axis=1, keepdims=True)
    ragg = jnp.dot(adj, emb_ref[...],
                   preferred_element_type=jnp.float32) / (rs + 1e-5)
    f = jnp.dot(fused_ref[...], w_ref[:H * D, :],
                preferred_element_type=jnp.float32)
    g = jnp.dot(ragg, w_ref[H * D:, :], preferred_element_type=jnp.float32)
    o_ref[...] = jnp.maximum(f + g + b_ref[...], 0.0)


def _relproj(rel_adj, rel_emb, fused, proj_w, proj_b2):
    bn = 1000
    return pl.pallas_call(
        _proj_body,
        grid=(N // bn,),
        in_specs=[pl.BlockSpec((bn, RN), lambda i: (i, 0)),
                  pl.BlockSpec((RN, D), lambda i: (0, 0)),
                  pl.BlockSpec((bn, H * D), lambda i: (i, 0)),
                  pl.BlockSpec((H * D + D, D), lambda i: (0, 0)),
                  pl.BlockSpec((1, D), lambda i: (0, 0))],
        out_specs=pl.BlockSpec((bn, D), lambda i: (i, 0)),
        out_shape=jax.ShapeDtypeStruct((N, D), jnp.float32),
    )(rel_adj, rel_emb, fused, proj_w, proj_b2)


def _forward(ent, rel_emb, rel_adj, edge, gat_w, gat_asrc, gat_adst,
             wq, wk, wv, proj_w, proj_b2):
    srcp = edge[0].astype(jnp.int32)
    dstp = edge[1].astype(jnp.int32)
    xs = [ent]
    x = ent
    for l in range(NLAYERS):
        hcat, al = _hidden_al(x, gat_w[l], gat_asrc[l], gat_adst[l])
        pay, wsum = _sc_edge_agg(hcat, al[:, 0], al[:, 2],
                                 al[:, 1], al[:, 3], srcp, dstp)
        x = _combine(pay, wsum)[:N]
        xs.append(x)
    fused = _mha(xs[0], xs[1], xs[2], wq, wk, wv)
    return _relproj(rel_adj, rel_emb, fused, proj_w, proj_b2)


def kernel(ent_sr, ent_tg, rel_emb_sr, rel_emb_tg, rel_adj_sr, rel_adj_tg,
           gat_W, gat_asrc, gat_adst, Wq, Wk, Wv, proj_W, proj_b,
           edge_sr, edge_tg):
    pb = proj_b.reshape(1, D)
    sr = _forward(ent_sr, rel_emb_sr, rel_adj_sr, edge_sr,
                  gat_W, gat_asrc, gat_adst, Wq, Wk, Wv, proj_W, pb)
    tg = _forward(ent_tg, rel_emb_tg, rel_adj_tg, edge_tg,
                  gat_W, gat_asrc, gat_adst, Wq, Wk, Wv, proj_W, pb)
    return (sr, tg)
